# baseline (speedup 1.0000x reference)
import functools

import jax
import jax.numpy as jnp
from jax import lax
from jax.experimental import pallas as pl
from jax.experimental.pallas import tpu as pltpu
from jax.experimental.pallas import tpu_sc as plsc

LANES = 16
NC, NS = 2, 16
NW = NC * NS


@functools.lru_cache(maxsize=None)
def _build_sc_kernel(N, L, D, V):
    TG = LANES
    GL = L * TG
    HL = D * TG
    RG = GL
    n_per_w = N // NW
    n_groups = n_per_w // TG
    n_pairs = n_groups // 2
    assert n_per_w * NW == N and n_pairs * 2 * TG == n_per_w

    mesh = plsc.VectorSubcoreMesh(core_axis_name="c", subcore_axis_name="s")

    @functools.partial(
        pl.kernel,
        mesh=mesh,
        compiler_params=pltpu.CompilerParams(
            needs_layout_passes=False),
        out_type=jax.ShapeDtypeStruct((N,), jnp.float32),
        scratch_types=[
            pltpu.VMEM((2 * GL,), jnp.int32),
            pltpu.VMEM((2 * GL,), jnp.float32),
            pltpu.VMEM((2 * HL,), jnp.float32),
            pltpu.VMEM((2 * RG, D), jnp.float32),
            pltpu.VMEM((GL,), jnp.float32),
            pltpu.VMEM((2 * TG,), jnp.float32),
            pltpu.SemaphoreType.DMA,
            pltpu.SemaphoreType.DMA,
            pltpu.SemaphoreType.DMA,
        ],
    )
    def body(xf, zf, hf, w, out, xs, zs, hs, rows, zmv, outv, sem_s, sem_r0, sem_r1):
        wid = lax.axis_index("c") * NS + lax.axis_index("s")
        lane = lax.iota(jnp.int32, LANES)
        base = wid * n_per_w

        def stage_copies(slot, tb):
            return (
                pltpu.make_async_copy(xf.at[pl.ds(tb * L, GL)],
                                      xs.at[pl.ds(slot * GL, GL)], sem_s),
                pltpu.make_async_copy(zf.at[pl.ds(tb * L, GL)],
                                      zs.at[pl.ds(slot * GL, GL)], sem_s),
                pltpu.make_async_copy(hf.at[pl.ds(tb * D, HL)],
                                      hs.at[pl.ds(slot * HL, HL)], sem_s),
            )

        def fire_stage(slot, tb):
            for c in stage_copies(slot, tb):
                c.start()

        def drain_stage(slot, tb):
            for c in stage_copies(slot, tb):
                c.wait()

        def row_copies(slot, tb):
            sem = sem_r0 if slot == 0 else sem_r1
            return [
                pltpu.make_async_copy(
                    w.at[xs.at[pl.ds(slot * GL + l * TG, TG)]],
                    rows.at[pl.ds(slot * RG + l * TG, TG)],
                    sem,
                )
                for l in range(L)
            ]

        def fire_rows(slot, tb):
            for c in row_copies(slot, tb):
                c.start()

        def drain_rows(slot, tb):
            for c in row_copies(slot, tb):
                c.wait()

        def prep_tail(slot):
            pc = jnp.ones((LANES,), jnp.float32)
            for l in range(L):
                xi = xs[pl.ds(slot * GL + l * TG, TG)]
                zl = zs[pl.ds(slot * GL + l * TG, TG)]
                m = xi != 0
                zmv[pl.ds(l * TG, TG)] = jnp.where(m, zl, 0.0)
                pc = pc * jnp.where(m, 1.0, 2.0)
            return pc

        def compute(slot, pc, out_half):
            rbase = slot * RG
            hbase = slot * HL
            perms = [jnp.bitwise_xor(lane, k) for k in (1, 2, 4, 8)]

            def tstep(tp, dots):
                hvs = [hs[pl.ds(hbase + tp * D + j * LANES, LANES)]
                       for j in range(D // LANES)]
                is_tp = lane == tp
                dots = list(dots)
                for l in range(L):
                    r = rbase + l * TG + tp
                    acc = None
                    for j in range(D // 32):
                        pk = plsc.bitcast(
                            rows[r, pl.ds(j * LANES, LANES)], jnp.bfloat16)
                        va, vb = plsc.unpack(
                            pk, format=plsc.PackFormat.INTERLEAVED)
                        t = va * hvs[2 * j] + vb * hvs[2 * j + 1]
                        acc = t if acc is None else acc + t
                    for p in perms:
                        acc = acc + jnp.take_along_axis(
                            acc, p, axis=0, mode="promise_in_bounds")
                    dots[l] = jnp.where(is_tp, acc, dots[l])
                return tuple(dots)

            zero = jnp.zeros((LANES,), jnp.float32)
            dots = lax.fori_loop(0, TG, tstep, tuple(zero for _ in range(L)))

            prod = pc
            for l in range(L):
                t = dots[l] * zmv[pl.ds(l * TG, TG)]
                prod = prod * (1.0 / (1.0 + jnp.exp(-t)))
            outv[pl.ds(out_half * TG, TG)] = prod

        fire_stage(0, base)
        drain_stage(0, base)
        fire_rows(0, base)
        fire_stage(1, base + TG)

        def pair(g2, carry):
            tb0 = base + g2 * (2 * TG)
            tb1 = tb0 + TG
            tb2 = tb0 + 2 * TG
            not_last = g2 < n_pairs - 1

            drain_stage(1, tb1)
            fire_rows(1, tb1)
            pc = prep_tail(0)
            drain_rows(0, tb0)
            compute(0, pc, 0)

            @pl.when(not_last)
            def _():
                fire_stage(0, tb2)

            @pl.when(not_last)
            def _():
                drain_stage(0, tb2)
                fire_rows(0, tb2)

            pc = prep_tail(1)
            drain_rows(1, tb1)
            compute(1, pc, 1)

            @pl.when(not_last)
            def _():
                fire_stage(1, tb2 + TG)

            pltpu.sync_copy(outv, out.at[pl.ds(tb0, 2 * TG)])
            return carry

        lax.fori_loop(0, n_pairs, pair, 0)

    return body


def kernel(x, z, h, w):
    B, T, L = x.shape
    D = h.shape[-1]
    N = B * T
    NG = N // LANES
    xf = x.reshape(NG, LANES, L).transpose(0, 2, 1).reshape(-1).astype(jnp.int32)
    zf = z.reshape(NG, LANES, L).transpose(0, 2, 1).reshape(-1).astype(jnp.float32)
    hf = (h.astype(jnp.float32)
          .reshape(N, D // 32, 16, 2)
          .transpose(0, 1, 3, 2)
          .reshape(-1))
    V = w.shape[0]
    packed = jax.lax.bitcast_convert_type(
        w.astype(jnp.bfloat16).reshape(V, D // 2, 2), jnp.float32)
    wp = jnp.concatenate(
        [packed, jnp.zeros((V, D - D // 2), jnp.float32)], axis=1)
    out = _build_sc_kernel(N, L, D, w.shape[0])(xf, zf, hf, wp)
    return out.reshape(B, T)


# speedup vs baseline: 1.4776x; 1.4776x over previous
import functools

import jax
import jax.numpy as jnp
from jax import lax
from jax.experimental import pallas as pl
from jax.experimental.pallas import tpu as pltpu
from jax.experimental.pallas import tpu_sc as plsc

LANES = 16
NC, NS = 2, 16
NW = NC * NS


@functools.lru_cache(maxsize=None)
def _build_sc_kernel(N, L, D, V):
    TG = LANES
    GL = L * TG
    HL = D * TG
    RG = GL
    n_per_w = N // NW
    n_groups = n_per_w // TG
    n_pairs = n_groups // 2
    assert n_per_w * NW == N and n_pairs * 2 * TG == n_per_w

    mesh = plsc.VectorSubcoreMesh(core_axis_name="c", subcore_axis_name="s")

    @functools.partial(
        pl.kernel,
        mesh=mesh,
        compiler_params=pltpu.CompilerParams(
            needs_layout_passes=False),
        out_type=jax.ShapeDtypeStruct((N,), jnp.float32),
        scratch_types=[
            pltpu.VMEM((2 * GL,), jnp.int32),
            pltpu.VMEM((2 * GL,), jnp.float32),
            pltpu.VMEM((2 * HL,), jnp.float32),
            pltpu.VMEM((2 * RG, D), jnp.float32),
            pltpu.VMEM((GL,), jnp.float32),
            pltpu.VMEM((2 * TG,), jnp.float32),
            pltpu.SemaphoreType.DMA,
            pltpu.SemaphoreType.DMA,
            pltpu.SemaphoreType.DMA,
        ],
    )
    def body(xf, zf, hf, w, out, xs, zs, hs, rows, zmv, outv, sem_s, sem_r0, sem_r1):
        wid = lax.axis_index("c") * NS + lax.axis_index("s")
        lane = lax.iota(jnp.int32, LANES)
        base = wid * n_per_w

        def stage_copies(slot, tb):
            return (
                pltpu.make_async_copy(xf.at[pl.ds(tb * L, GL)],
                                      xs.at[pl.ds(slot * GL, GL)], sem_s),
                pltpu.make_async_copy(zf.at[pl.ds(tb * L, GL)],
                                      zs.at[pl.ds(slot * GL, GL)], sem_s),
                pltpu.make_async_copy(hf.at[pl.ds(tb * D, HL)],
                                      hs.at[pl.ds(slot * HL, HL)], sem_s),
            )

        def fire_stage(slot, tb):
            for c in stage_copies(slot, tb):
                c.start()

        def drain_stage(slot, tb):
            for c in stage_copies(slot, tb):
                c.wait()

        def row_copies(slot, tb):
            sem = sem_r0 if slot == 0 else sem_r1
            return [
                pltpu.make_async_copy(
                    w.at[xs.at[pl.ds(slot * GL + l * TG, TG)]],
                    rows.at[pl.ds(slot * RG + l * TG, TG)],
                    sem,
                )
                for l in range(L)
            ]

        def fire_rows(slot, tb):
            for c in row_copies(slot, tb):
                c.start()

        def drain_rows(slot, tb):
            for c in row_copies(slot, tb):
                c.wait()

        def prep_tail(slot):
            pc = jnp.ones((LANES,), jnp.float32)
            for l in range(L):
                xi = xs[pl.ds(slot * GL + l * TG, TG)]
                zl = zs[pl.ds(slot * GL + l * TG, TG)]
                m = xi != 0
                zmv[pl.ds(l * TG, TG)] = jnp.where(m, zl, 0.0)
                pc = pc * jnp.where(m, 1.0, 2.0)
            return pc

        def compute(slot, pc, out_half):
            rbase = slot * RG
            hbase = slot * HL
            perms = [jnp.bitwise_xor(lane, k) for k in (1, 2, 4, 8)]

            def tstep(tp, dots):
                hvs = [hs[pl.ds(hbase + tp * D + j * LANES, LANES)]
                       for j in range(D // LANES)]
                is_tp = lane == tp
                dots = list(dots)
                for l in range(L):
                    r = rbase + l * TG + tp
                    acc = None
                    for j in range(D // 32):
                        pk = plsc.bitcast(
                            rows[r, pl.ds(j * LANES, LANES)], jnp.bfloat16)
                        va, vb = plsc.unpack(
                            pk, format=plsc.PackFormat.INTERLEAVED)
                        t = va * hvs[2 * j] + vb * hvs[2 * j + 1]
                        acc = t if acc is None else acc + t
                    for p in perms:
                        acc = acc + jnp.take_along_axis(
                            acc, p, axis=0, mode="promise_in_bounds")
                    dots[l] = jnp.where(is_tp, acc, dots[l])
                return tuple(dots)

            zero = jnp.zeros((LANES,), jnp.float32)
            dots = lax.fori_loop(0, TG, tstep, tuple(zero for _ in range(L)))

            prod = pc
            for l in range(L):
                t = dots[l] * zmv[pl.ds(l * TG, TG)]
                prod = prod * (1.0 / (1.0 + jnp.exp(-t)))
            outv[pl.ds(out_half * TG, TG)] = prod

        fire_stage(0, base)
        drain_stage(0, base)
        fire_rows(0, base)
        fire_stage(1, base + TG)

        def pair(g2, carry):
            tb0 = base + g2 * (2 * TG)
            tb1 = tb0 + TG
            tb2 = tb0 + 2 * TG
            not_last = g2 < n_pairs - 1

            drain_stage(1, tb1)
            fire_rows(1, tb1)
            pc = prep_tail(0)
            drain_rows(0, tb0)
            compute(0, pc, 0)

            @pl.when(not_last)
            def _():
                fire_stage(0, tb2)

            @pl.when(not_last)
            def _():
                drain_stage(0, tb2)
                fire_rows(0, tb2)

            pc = prep_tail(1)
            drain_rows(1, tb1)
            compute(1, pc, 1)

            @pl.when(not_last)
            def _():
                fire_stage(1, tb2 + TG)

            pltpu.sync_copy(outv, out.at[pl.ds(tb0, 2 * TG)])
            return carry

        lax.fori_loop(0, n_pairs, pair, 0)

    return body


def kernel(x, z, h, w):
    B, T, L = x.shape
    D = h.shape[-1]
    N = B * T
    NG = N // LANES
    xf = x.reshape(NG, LANES, L).transpose(0, 2, 1).reshape(-1).astype(jnp.int32)
    zf = z.reshape(NG, LANES, L).transpose(0, 2, 1).reshape(-1).astype(jnp.float32)
    hf = (h.astype(jnp.float32)
          .reshape(N, 2, D // 32, 16)
          .transpose(0, 2, 1, 3)
          .reshape(-1))
    V = w.shape[0]
    wu = jax.lax.bitcast_convert_type(w.astype(jnp.float32), jnp.uint32)
    wr = (wu + jnp.uint32(0x7FFF) + ((wu >> 16) & jnp.uint32(1))) >> 16
    packed = wr[:, : D // 2] | (wr[:, D // 2 :] << 16)
    wp = jax.lax.bitcast_convert_type(
        jnp.concatenate(
            [packed, jnp.zeros((V, D - D // 2), jnp.uint32)], axis=1),
        jnp.float32)
    out = _build_sc_kernel(N, L, D, w.shape[0])(xf, zf, hf, wp)
    return out.reshape(B, T)


# speedup vs baseline: 4.7373x; 3.2061x over previous
import functools

import jax
import jax.numpy as jnp
from jax import lax
from jax.experimental import pallas as pl
from jax.experimental.pallas import tpu as pltpu
from jax.experimental.pallas import tpu_sc as plsc

LANES = 16
NC, NS = 2, 16
NW = NC * NS


@functools.lru_cache(maxsize=None)
def _build_sc_kernel(N, L, D, V):
    TG = LANES
    GL = L * TG
    HL = D * TG
    RG = GL
    n_per_w = N // NW
    n_groups = n_per_w // TG
    n_pairs = n_groups // 2
    assert n_per_w * NW == N and n_pairs * 2 * TG == n_per_w

    mesh = plsc.VectorSubcoreMesh(core_axis_name="c", subcore_axis_name="s")

    @functools.partial(
        pl.kernel,
        mesh=mesh,
        compiler_params=pltpu.CompilerParams(
            needs_layout_passes=False),
        out_type=[jax.ShapeDtypeStruct((N // 2,), jnp.float32),
                  jax.ShapeDtypeStruct((N // 2,), jnp.float32)],
        scratch_types=[
            pltpu.VMEM((2 * GL,), jnp.int32),
            pltpu.VMEM((2 * GL,), jnp.float32),
            pltpu.VMEM((2 * HL,), jnp.float32),
            pltpu.VMEM((2 * RG, D), jnp.float32),
            pltpu.VMEM((GL,), jnp.float32),
            pltpu.VMEM((2 * TG,), jnp.float32),
            pltpu.SemaphoreType.DMA,
            pltpu.SemaphoreType.DMA,
            pltpu.SemaphoreType.DMA,
        ],
    )
    def body(xf, zf, hf, w, out0, out1, xs, zs, hs, rows, zmv, outv,
             sem_s, sem_r0, sem_r1):
        cid = lax.axis_index("c")
        wid = cid * NS + lax.axis_index("s")
        lane = lax.iota(jnp.int32, LANES)
        base = wid * n_per_w

        def stage_copies(slot, tb):
            return (
                pltpu.make_async_copy(xf.at[pl.ds(tb * L, GL)],
                                      xs.at[pl.ds(slot * GL, GL)], sem_s),
                pltpu.make_async_copy(zf.at[pl.ds(tb * L, GL)],
                                      zs.at[pl.ds(slot * GL, GL)], sem_s),
                pltpu.make_async_copy(hf.at[pl.ds(tb * D, HL)],
                                      hs.at[pl.ds(slot * HL, HL)], sem_s),
            )

        def fire_stage(slot, tb):
            for c in stage_copies(slot, tb):
                c.start()

        def drain_stage(slot, tb):
            for c in stage_copies(slot, tb):
                c.wait()

        def row_copies(slot, tb):
            sem = sem_r0 if slot == 0 else sem_r1
            return [
                pltpu.make_async_copy(
                    w.at[xs.at[pl.ds(slot * GL + l * TG, TG)]],
                    rows.at[pl.ds(slot * RG + l * TG, TG)],
                    sem,
                )
                for l in range(L)
            ]

        def fire_rows(slot, tb):
            for c in row_copies(slot, tb):
                c.start()

        def drain_rows(slot, tb):
            for c in row_copies(slot, tb):
                c.wait()

        def prep_tail(slot):
            pc = jnp.ones((LANES,), jnp.float32)
            for l in range(L):
                xi = xs[pl.ds(slot * GL + l * TG, TG)]
                zl = zs[pl.ds(slot * GL + l * TG, TG)]
                m = xi != 0
                zmv[pl.ds(l * TG, TG)] = jnp.where(m, zl, 0.0)
                pc = pc * jnp.where(m, 1.0, 2.0)
            return pc

        def compute(slot, pc, out_half):
            rbase = slot * RG
            hbase = slot * HL
            perms = [jnp.bitwise_xor(lane, k) for k in (1, 2, 4, 8)]

            def tstep(tp, dots):
                hvs = [hs[pl.ds(hbase + tp * D + j * LANES, LANES)]
                       for j in range(D // LANES)]
                is_tp = lane == tp
                dots = list(dots)
                for l in range(L):
                    r = rbase + l * TG + tp
                    acc = rows[r, pl.ds(0, LANES)] * hvs[0]
                    for j in range(1, D // LANES):
                        acc = acc + rows[r, pl.ds(j * LANES, LANES)] * hvs[j]
                    for p in perms:
                        acc = acc + jnp.take_along_axis(
                            acc, p, axis=0, mode="promise_in_bounds")
                    dots[l] = jnp.where(is_tp, acc, dots[l])
                return tuple(dots)

            zero = jnp.zeros((LANES,), jnp.float32)
            dots = lax.fori_loop(0, TG, tstep, tuple(zero for _ in range(L)))

            prod = pc
            for l in range(L):
                t = dots[l] * zmv[pl.ds(l * TG, TG)]
                prod = prod * (1.0 / (1.0 + jnp.exp(-t)))
            outv[pl.ds(out_half * TG, TG)] = prod

        fire_stage(0, base)
        drain_stage(0, base)
        fire_rows(0, base)
        fire_stage(1, base + TG)

        def pair(g2, carry):
            tb0 = base + g2 * (2 * TG)
            tb1 = tb0 + TG
            tb2 = tb0 + 2 * TG
            not_last = g2 < n_pairs - 1

            drain_stage(1, tb1)
            fire_rows(1, tb1)
            pc = prep_tail(0)
            drain_rows(0, tb0)
            compute(0, pc, 0)

            @pl.when(not_last)
            def _():
                fire_stage(0, tb2)

            @pl.when(not_last)
            def _():
                drain_stage(0, tb2)
                fire_rows(0, tb2)

            pc = prep_tail(1)
            drain_rows(1, tb1)
            compute(1, pc, 1)

            @pl.when(not_last)
            def _():
                fire_stage(1, tb2 + TG)

            @pl.when(cid == 0)
            def _():
                pltpu.sync_copy(outv, out0.at[pl.ds(tb0, 2 * TG)])

            @pl.when(cid == 1)
            def _():
                pltpu.sync_copy(outv, out1.at[pl.ds(tb0 - N // 2, 2 * TG)])

            return carry

        lax.fori_loop(0, n_pairs, pair, 0)

    return body


def kernel(x, z, h, w):
    B, T, L = x.shape
    D = h.shape[-1]
    N = B * T
    NG = N // LANES
    xf = x.reshape(NG, LANES, L).transpose(0, 2, 1).reshape(-1).astype(jnp.int32)
    zf = z.reshape(NG, LANES, L).transpose(0, 2, 1).reshape(-1).astype(jnp.float32)
    hf = h.reshape(-1).astype(jnp.float32)
    wp = w.astype(jnp.float32)
    out0, out1 = _build_sc_kernel(N, L, D, w.shape[0])(xf, zf, hf, wp)
    return jnp.concatenate([out0, out1]).reshape(B, T)


# speedup vs baseline: 4.7615x; 1.0051x over previous
import functools

import jax
import jax.numpy as jnp
from jax import lax
from jax.experimental import pallas as pl
from jax.experimental.pallas import tpu as pltpu
from jax.experimental.pallas import tpu_sc as plsc

LANES = 16
NC, NS = 2, 16
NW = NC * NS


@functools.lru_cache(maxsize=None)
def _build_sc_kernel(N, L, D, V):
    TG = LANES
    RG = L * TG
    HL = D * TG
    n_per_w = N // NW
    n_groups = n_per_w // TG
    n_pairs = n_groups // 2
    assert n_per_w * NW == N and n_pairs * 2 * TG == n_per_w
    assert L == LANES + 1

    mesh = plsc.VectorSubcoreMesh(core_axis_name="c", subcore_axis_name="s")

    @functools.partial(
        pl.kernel,
        mesh=mesh,
        compiler_params=pltpu.CompilerParams(needs_layout_passes=False),
        out_type=jax.ShapeDtypeStruct((N,), jnp.float32),
        scratch_types=[
            pltpu.VMEM((2 * TG, L), jnp.int32),
            pltpu.VMEM((2 * TG, L), jnp.float32),
            pltpu.VMEM((2 * HL,), jnp.float32),
            pltpu.VMEM((2 * RG, D), jnp.float32),
            pltpu.VMEM((2 * TG,), jnp.int32),
            pltpu.VMEM((2 * TG,), jnp.float32),
            pltpu.SemaphoreType.DMA,
            pltpu.SemaphoreType.DMA,
            pltpu.SemaphoreType.DMA,
        ],
    )
    def body(xn, zn, hf, w, out, xs, zs, hs, rows, vidx, outv,
             sem_s, sem_r0, sem_r1):
        wid = lax.axis_index("c") * NS + lax.axis_index("s")
        lane = lax.iota(jnp.int32, LANES)
        base = wid * n_per_w

        def stage_copies(slot, tb):
            return (
                pltpu.make_async_copy(xn.at[pl.ds(tb, TG), :],
                                      xs.at[pl.ds(slot * TG, TG), :], sem_s),
                pltpu.make_async_copy(zn.at[pl.ds(tb, TG), :],
                                      zs.at[pl.ds(slot * TG, TG), :], sem_s),
                pltpu.make_async_copy(hf.at[pl.ds(tb * D, HL)],
                                      hs.at[pl.ds(slot * HL, HL)], sem_s),
            )

        def fire_stage(slot, tb):
            for c in stage_copies(slot, tb):
                c.start()

        def drain_stage(slot, tb):
            for c in stage_copies(slot, tb):
                c.wait()

        def row_copies(slot, tb):
            sem = sem_r0 if slot == 0 else sem_r1
            cps = [
                pltpu.make_async_copy(
                    w.at[xs.at[slot * TG + t, pl.ds(0, LANES)]],
                    rows.at[pl.ds(slot * RG + t * LANES, LANES)],
                    sem,
                )
                for t in range(TG)
            ]
            cps.append(
                pltpu.make_async_copy(
                    w.at[vidx.at[pl.ds(slot * TG, TG)]],
                    rows.at[pl.ds(slot * RG + TG * LANES, TG)],
                    sem,
                )
            )
            return cps

        def fire_rows(slot, tb):
            x17 = plsc.load_gather(
                xs, [slot * TG + lane, jnp.full((LANES,), L - 1, jnp.int32)])
            vidx[pl.ds(slot * TG, TG)] = x17
            for c in row_copies(slot, tb):
                c.start()

        def drain_rows(slot, tb):
            for c in row_copies(slot, tb):
                c.wait()

        def compute(slot, out_half):
            rbase = slot * RG
            hbase = slot * HL
            perms = [jnp.bitwise_xor(lane, k) for k in (1, 2, 4, 8)]

            def tstep(tp, acc_out):
                hvs = [hs[pl.ds(hbase + tp * D + j * LANES, LANES)]
                       for j in range(D // LANES)]
                rb = rbase + tp * LANES
                dots = jnp.zeros((LANES,), jnp.float32)
                for s in range(LANES):
                    acc = rows[rb + s, pl.ds(0, LANES)] * hvs[0]
                    for j in range(1, D // LANES):
                        acc = acc + rows[rb + s, pl.ds(j * LANES, LANES)] * hvs[j]
                    for p in perms:
                        acc = acc + jnp.take_along_axis(
                            acc, p, axis=0, mode="promise_in_bounds")
                    dots = jnp.where(lane == s, acc, dots)
                zrow = zs[slot * TG + tp, pl.ds(0, LANES)]
                xrow = xs[slot * TG + tp, pl.ds(0, LANES)]
                y = 1.0 / (1.0 + jnp.exp(-dots * zrow))
                y = jnp.where(xrow != 0, y, 1.0)
                for p in perms:
                    y = y * jnp.take_along_axis(
                        y, p, axis=0, mode="promise_in_bounds")
                return jnp.where(lane == tp, y, acc_out)

            prod = lax.fori_loop(0, TG, tstep, jnp.ones((LANES,), jnp.float32))

            vb = rbase + TG * LANES
            dots17 = jnp.zeros((LANES,), jnp.float32)
            for t in range(TG):
                acc = rows[vb + t, pl.ds(0, LANES)] * hs[pl.ds(hbase + t * D, LANES)]
                for j in range(1, D // LANES):
                    acc = acc + (rows[vb + t, pl.ds(j * LANES, LANES)]
                                 * hs[pl.ds(hbase + t * D + j * LANES, LANES)])
                for p in perms:
                    acc = acc + jnp.take_along_axis(
                        acc, p, axis=0, mode="promise_in_bounds")
                dots17 = jnp.where(lane == t, acc, dots17)
            z17 = plsc.load_gather(
                zs, [slot * TG + lane, jnp.full((LANES,), L - 1, jnp.int32)])
            x17 = vidx[pl.ds(slot * TG, TG)]
            y17 = 1.0 / (1.0 + jnp.exp(-dots17 * z17))
            y17 = jnp.where(x17 != 0, y17, 1.0)
            outv[pl.ds(out_half * TG, TG)] = prod * y17

        fire_stage(0, base)
        drain_stage(0, base)
        fire_rows(0, base)
        fire_stage(1, base + TG)

        def pair(g2, carry):
            tb0 = base + g2 * (2 * TG)
            tb1 = tb0 + TG
            tb2 = tb0 + 2 * TG
            not_last = g2 < n_pairs - 1

            drain_stage(1, tb1)
            fire_rows(1, tb1)
            drain_rows(0, tb0)
            compute(0, 0)

            @pl.when(not_last)
            def _():
                fire_stage(0, tb2)

            @pl.when(not_last)
            def _():
                drain_stage(0, tb2)
                fire_rows(0, tb2)

            drain_rows(1, tb1)
            compute(1, 1)

            @pl.when(not_last)
            def _():
                fire_stage(1, tb2 + TG)

            pltpu.sync_copy(outv, out.at[pl.ds(tb0, 2 * TG)])
            return carry

        lax.fori_loop(0, n_pairs, pair, 0)

    return body


def kernel(x, z, h, w):
    B, T, L = x.shape
    D = h.shape[-1]
    N = B * T
    xn = x.reshape(N, L).astype(jnp.int32)
    zn = z.reshape(N, L).astype(jnp.float32)
    hf = h.reshape(-1).astype(jnp.float32)
    out = _build_sc_kernel(N, L, D, w.shape[0])(xn, zn, hf, w.astype(jnp.float32))
    return out.reshape(B, T)
